# Initial kernel scaffold; baseline (speedup 1.0000x reference)
import jax
import jax.numpy as jnp
from jax.experimental import pallas as pl


def kernel(x, adj_indices, adj_values, W, a1, a2):
    raise NotImplementedError("write your pallas kernel here")


# speedup vs baseline: 9.8322x; 9.8322x over previous
import functools

import jax
import jax.numpy as jnp
from jax import lax
from jax.experimental import pallas as pl
from jax.experimental.pallas import tpu as pltpu
from jax.experimental.pallas import tpu_sc as plsc

N = 10000
E = 320000
D = 128
NC = 2
NS = 16
NW = NC * NS
EPW = E // NW
CHUNK = 80
NCHUNK = EPW // CHUNK
NPAD = 10240
ROWS_PER_TILE = NPAD // NS
SNR = NPAD // D


def _stage1_body(x_ref, w_ref, a2_ref, tab_ref, rd_ref):
    s = jnp.dot(x_ref[...], w_ref[...], preferred_element_type=jnp.float32)
    rd = jax.nn.sigmoid(
        jnp.dot(s, a2_ref[...], preferred_element_type=jnp.float32))
    tab_ref[...] = rd * s
    rd_ref[...] = rd


def _stage1(x, W, a2):
    B = 2000
    return pl.pallas_call(
        _stage1_body,
        grid=(N // B,),
        in_specs=[
            pl.BlockSpec((B, D), lambda i: (i, 0)),
            pl.BlockSpec((D, D), lambda i: (0, 0)),
            pl.BlockSpec((D, 1), lambda i: (0, 0)),
        ],
        out_specs=[
            pl.BlockSpec((B, D), lambda i: (i, 0)),
            pl.BlockSpec((B, 1), lambda i: (i, 0)),
        ],
        out_shape=[
            jax.ShapeDtypeStruct((N, D), jnp.float32),
            jax.ShapeDtypeStruct((N, 1), jnp.float32),
        ],
    )(x, W, a2)


def _sc_body(tab_hbm, rd_hbm, col_hbm, row_hbm, val_hbm, out_hbm, sn_hbm,
             colv, rowv, valv, rows_v, rd_v, snorm_v, acc, sem):
    c = lax.axis_index("c")
    s = lax.axis_index("s")
    w = s * NC + c

    zf16 = jnp.zeros((16,), jnp.float32)

    def zero_body(i, carry):
        for j in range(D // 16):
            rows_v[i, pl.ds(j * 16, 16)] = zf16
        return carry
    lax.fori_loop(0, CHUNK, zero_body, 0, unroll=False)
    for b in range(ROWS_PER_TILE // CHUNK):
        pltpu.sync_copy(rows_v,
                        acc.at[pl.ds(s * ROWS_PER_TILE + b * CHUNK, CHUNK)])

    def zero_sn(i, carry):
        for j in range(D // 16):
            snorm_v[i, pl.ds(j * 16, 16)] = zf16
        return carry
    lax.fori_loop(0, SNR, zero_sn, 0, unroll=False)

    pltpu.sync_copy(rd_hbm, rd_v)
    plsc.subcore_barrier()

    base0 = w * EPW

    def chunk_body(k, carry):
        base = base0 + k * CHUNK
        pltpu.sync_copy(col_hbm.at[pl.ds(base, CHUNK)], colv)
        pltpu.sync_copy(row_hbm.at[pl.ds(base, CHUNK)], rowv)
        pltpu.sync_copy(val_hbm.at[pl.ds(base, CHUNK)], valv)
        pltpu.async_copy(tab_hbm.at[colv], rows_v, sem).wait()

        def group_body(g, carry2):
            sl16 = pl.ds(g * 16, 16)
            val16 = valv[sl16]
            col16 = colv[sl16]
            row16 = rowv[sl16]
            rd16 = plsc.load_gather(rd_v, [col16])
            plsc.addupdate_scatter(
                snorm_v,
                [lax.shift_right_logical(row16, 7),
                 lax.bitwise_and(row16, 127)],
                val16 * rd16)
            for l in range(16):
                v = val16[l]
                e = g * 16 + l
                for j in range(D // 16):
                    sl = pl.ds(j * 16, 16)
                    rows_v[e, sl] = rows_v[e, sl] * v
            return carry2

        lax.fori_loop(0, CHUNK // 16, group_body, 0, unroll=False)
        pltpu.sync_copy(rows_v, acc.at[rowv], add=True)
        return carry

    lax.fori_loop(0, NCHUNK, chunk_body, 0, unroll=False)
    pltpu.sync_copy(snorm_v, sn_hbm.at[w])
    plsc.subcore_barrier()

    pltpu.sync_copy(acc.at[pl.ds(s * ROWS_PER_TILE, ROWS_PER_TILE)],
                    out_hbm.at[c, pl.ds(s * ROWS_PER_TILE, ROWS_PER_TILE)])


_sc_kernel = functools.partial(
    pl.kernel,
    out_type=[
        jax.ShapeDtypeStruct((NC, NPAD, D), jnp.float32),
        jax.ShapeDtypeStruct((NW, SNR, D), jnp.float32),
    ],
    mesh=plsc.VectorSubcoreMesh(core_axis_name="c", subcore_axis_name="s"),
    compiler_params=pltpu.CompilerParams(needs_layout_passes=False),
    scratch_types=[
        pltpu.VMEM((CHUNK,), jnp.int32),
        pltpu.VMEM((CHUNK,), jnp.int32),
        pltpu.VMEM((CHUNK,), jnp.float32),
        pltpu.VMEM((CHUNK, D), jnp.float32),
        pltpu.VMEM((NPAD,), jnp.float32),
        pltpu.VMEM((SNR, D), jnp.float32),
        pltpu.VMEM_SHARED((NPAD, D), jnp.float32),
        pltpu.SemaphoreType.DMA,
    ],
)(_sc_body)


_B3 = 2048


def _stage3_body(acc_ref, sn_ref, a1_ref, out_ref):
    a = acc_ref[0] + acc_ref[1]
    sn_sum = jnp.sum(sn_ref[...], axis=0)
    rows = lax.broadcasted_iota(jnp.int32, (_B3, _B3 // D), 0)
    cols = lax.broadcasted_iota(jnp.int32, (_B3, _B3 // D), 1)
    b1 = (rows // D == cols).astype(jnp.float32)
    tmp = jnp.dot(b1, sn_sum, preferred_element_type=jnp.float32)
    rows2 = lax.broadcasted_iota(jnp.int32, (_B3, D), 0)
    cols2 = lax.broadcasted_iota(jnp.int32, (_B3, D), 1)
    sel = (rows2 % D == cols2).astype(jnp.float32)
    snb = jnp.dot(tmp * sel, jnp.ones((D, D), jnp.float32),
                  preferred_element_type=jnp.float32)
    out = a / snb
    l_d = jax.nn.softplus(
        jnp.dot(out, a1_ref[...], preferred_element_type=jnp.float32))
    out = jnp.maximum(out, 0.0) + l_d * jnp.minimum(out, 0.0)
    out_ref[...] = jnp.where(out > 0, out,
                             jnp.exp(jnp.minimum(out, 0.0)) - 1.0)


def _stage3(acc, sn, a1):
    return pl.pallas_call(
        _stage3_body,
        grid=(NPAD // _B3,),
        in_specs=[
            pl.BlockSpec((NC, _B3, D), lambda i: (0, i, 0)),
            pl.BlockSpec((NW, _B3 // D, D), lambda i: (0, i, 0)),
            pl.BlockSpec((D, 1), lambda i: (0, 0)),
        ],
        out_specs=pl.BlockSpec((_B3, D), lambda i: (i, 0)),
        out_shape=jax.ShapeDtypeStruct((NPAD, D), jnp.float32),
    )(acc, sn, a1)


def kernel(x, adj_indices, adj_values, W, a1, a2):
    row = adj_indices[0].astype(jnp.int32)
    col = adj_indices[1].astype(jnp.int32)
    val = adj_values.astype(jnp.float32)
    tab, rd = _stage1(x, W, a2)
    rd1 = jnp.pad(rd[:, 0], (0, NPAD - N))
    acc, sn = _sc_kernel(tab, rd1, col, row, val)
    return _stage3(acc, sn, a1)[:N]


# speedup vs baseline: 16.6053x; 1.6889x over previous
import functools

import jax
import jax.numpy as jnp
from jax import lax
from jax.experimental import pallas as pl
from jax.experimental.pallas import tpu as pltpu
from jax.experimental.pallas import tpu_sc as plsc

N = 10000
E = 320000
D = 128
NC = 2
NS = 16
NW = NC * NS
EPW = E // NW
CHUNK = 80
NCHUNK = EPW // CHUNK
SBC = 25
NSB = NCHUNK // SBC
NPAIR = SBC // 2
CVW = 3 * CHUNK
NPAD = 10240
ROWS_PER_TILE = NPAD // NS
SNR = NPAD // D


def _stage1_body(x_ref, w_ref, a2_ref, tab_ref, rd_ref):
    s = jnp.dot(x_ref[...], w_ref[...], preferred_element_type=jnp.float32)
    rd = jax.nn.sigmoid(
        jnp.dot(s, a2_ref[...], preferred_element_type=jnp.float32))
    tab_ref[...] = rd * s
    rd_ref[...] = rd


def _stage1(x, W, a2):
    B = 2000
    return pl.pallas_call(
        _stage1_body,
        grid=(N // B,),
        in_specs=[
            pl.BlockSpec((B, D), lambda i: (i, 0)),
            pl.BlockSpec((D, D), lambda i: (0, 0)),
            pl.BlockSpec((D, 1), lambda i: (0, 0)),
        ],
        out_specs=[
            pl.BlockSpec((B, D), lambda i: (i, 0)),
            pl.BlockSpec((B, 1), lambda i: (i, 0)),
        ],
        out_shape=[
            jax.ShapeDtypeStruct((N, D), jnp.float32),
            jax.ShapeDtypeStruct((N, 1), jnp.float32),
        ],
    )(x, W, a2)


def _sc_body(tab_hbm, rd_hbm, cv_hbm, out_hbm, sn_hbm,
             cv_s, rowv0, rowv1, rows0, rows1,
             rd_v, snorm_v, acc, sg0, sg1, ss0, ss1):
    c = lax.axis_index("c")
    s = lax.axis_index("s")
    w = s * NC + c

    zf16 = jnp.zeros((16,), jnp.float32)

    def zero_body(i, carry):
        for j in range(D // 16):
            rows0[i, pl.ds(j * 16, 16)] = zf16
        return carry
    lax.fori_loop(0, CHUNK, zero_body, 0, unroll=False)
    for b in range(ROWS_PER_TILE // CHUNK):
        pltpu.sync_copy(rows0,
                        acc.at[pl.ds(s * ROWS_PER_TILE + b * CHUNK, CHUNK)])

    def zero_sn(i, carry):
        for j in range(D // 16):
            snorm_v[i, pl.ds(j * 16, 16)] = zf16
        return carry
    lax.fori_loop(0, SNR, zero_sn, 0, unroll=False)

    pltpu.sync_copy(rd_hbm, rd_v)
    plsc.subcore_barrier()

    def start_gather(kk, rows_b, sem):
        pltpu.async_copy(tab_hbm.at[cv_s.at[pl.ds(kk * CVW, CHUNK)]],
                         rows_b, sem)

    def wait_gather(kk, rows_b, sem):
        pltpu.make_async_copy(tab_hbm.at[cv_s.at[pl.ds(kk * CVW, CHUNK)]],
                              rows_b, sem).wait()

    def start_scatter(rows_b, rowv_b, sem):
        pltpu.async_copy(rows_b, acc.at[rowv_b], sem, add=True)

    def wait_scatter(rows_b, rowv_b, sem):
        pltpu.make_async_copy(rows_b, acc.at[rowv_b], sem).wait()

    def process(kk, rows_b, rowv_b):
        def group_body(g, carry2):
            base = kk * CVW + g * 16
            col16 = cv_s[pl.ds(base, 16)]
            row16 = cv_s[pl.ds(base + CHUNK, 16)]
            val16 = plsc.bitcast(cv_s[pl.ds(base + 2 * CHUNK, 16)],
                                 jnp.float32)
            rowv_b[pl.ds(g * 16, 16)] = row16
            rd16 = plsc.load_gather(rd_v, [col16])
            plsc.addupdate_scatter(
                snorm_v,
                [lax.shift_right_logical(row16, 7),
                 lax.bitwise_and(row16, 127)],
                val16 * rd16)
            for l in range(16):
                v = val16[l]
                e = g * 16 + l
                for j in range(D // 16):
                    sl = pl.ds(j * 16, 16)
                    rows_b[e, sl] = rows_b[e, sl] * v
            return carry2

        lax.fori_loop(0, CHUNK // 16, group_body, 0, unroll=False)

    def sb_body(sb, carry):
        pltpu.sync_copy(cv_hbm.at[pl.ds((w * NSB + sb) * (SBC * CVW),
                                        SBC * CVW)], cv_s)
        start_gather(0, rows0, sg0)

        def pair_body(p, carry2):
            ke = 2 * p
            ko = ke + 1
            kn = ke + 2
            wait_gather(ke, rows0, sg0)

            @pl.when(p > 0)
            def _():
                wait_scatter(rows1, rowv1, ss1)
            start_gather(ko, rows1, sg1)

            process(ke, rows0, rowv0)
            start_scatter(rows0, rowv0, ss0)

            wait_gather(ko, rows1, sg1)
            process(ko, rows1, rowv1)
            start_scatter(rows1, rowv1, ss1)
            wait_scatter(rows0, rowv0, ss0)
            start_gather(kn, rows0, sg0)
            return carry2

        lax.fori_loop(0, NPAIR, pair_body, 0, unroll=False)

        wait_gather(SBC - 1, rows0, sg0)
        process(SBC - 1, rows0, rowv0)
        start_scatter(rows0, rowv0, ss0)
        wait_scatter(rows1, rowv1, ss1)
        wait_scatter(rows0, rowv0, ss0)
        return carry

    lax.fori_loop(0, NSB, sb_body, 0, unroll=False)

    pltpu.sync_copy(snorm_v, sn_hbm.at[w])
    plsc.subcore_barrier()

    pltpu.sync_copy(acc.at[pl.ds(s * ROWS_PER_TILE, ROWS_PER_TILE)],
                    out_hbm.at[c, pl.ds(s * ROWS_PER_TILE, ROWS_PER_TILE)])


_sc_kernel = functools.partial(
    pl.kernel,
    out_type=[
        jax.ShapeDtypeStruct((NC, NPAD, D), jnp.float32),
        jax.ShapeDtypeStruct((NW, SNR, D), jnp.float32),
    ],
    mesh=plsc.VectorSubcoreMesh(core_axis_name="c", subcore_axis_name="s"),
    compiler_params=pltpu.CompilerParams(needs_layout_passes=False),
    scratch_types=[
        pltpu.VMEM((SBC * CVW,), jnp.int32),
        pltpu.VMEM((CHUNK,), jnp.int32),
        pltpu.VMEM((CHUNK,), jnp.int32),
        pltpu.VMEM((CHUNK, D), jnp.float32),
        pltpu.VMEM((CHUNK, D), jnp.float32),
        pltpu.VMEM((NPAD,), jnp.float32),
        pltpu.VMEM((SNR, D), jnp.float32),
        pltpu.VMEM_SHARED((NPAD, D), jnp.float32),
        pltpu.SemaphoreType.DMA,
        pltpu.SemaphoreType.DMA,
        pltpu.SemaphoreType.DMA,
        pltpu.SemaphoreType.DMA,
    ],
)(_sc_body)


_B3 = 2048


def _stage3_body(acc_ref, sn_ref, a1_ref, out_ref):
    a = acc_ref[0] + acc_ref[1]
    sn_sum = jnp.sum(sn_ref[...], axis=0)
    rows = lax.broadcasted_iota(jnp.int32, (_B3, _B3 // D), 0)
    cols = lax.broadcasted_iota(jnp.int32, (_B3, _B3 // D), 1)
    b1 = (rows // D == cols).astype(jnp.float32)
    tmp = jnp.dot(b1, sn_sum, preferred_element_type=jnp.float32)
    rows2 = lax.broadcasted_iota(jnp.int32, (_B3, D), 0)
    cols2 = lax.broadcasted_iota(jnp.int32, (_B3, D), 1)
    sel = (rows2 % D == cols2).astype(jnp.float32)
    snb = jnp.dot(tmp * sel, jnp.ones((D, D), jnp.float32),
                  preferred_element_type=jnp.float32)
    out = a / snb
    l_d = jax.nn.softplus(
        jnp.dot(out, a1_ref[...], preferred_element_type=jnp.float32))
    out = jnp.maximum(out, 0.0) + l_d * jnp.minimum(out, 0.0)
    out_ref[...] = jnp.where(out > 0, out,
                             jnp.exp(jnp.minimum(out, 0.0)) - 1.0)


def _stage3(acc, sn, a1):
    return pl.pallas_call(
        _stage3_body,
        grid=(NPAD // _B3,),
        in_specs=[
            pl.BlockSpec((NC, _B3, D), lambda i: (0, i, 0)),
            pl.BlockSpec((NW, _B3 // D, D), lambda i: (0, i, 0)),
            pl.BlockSpec((D, 1), lambda i: (0, 0)),
        ],
        out_specs=pl.BlockSpec((_B3, D), lambda i: (i, 0)),
        out_shape=jax.ShapeDtypeStruct((NPAD, D), jnp.float32),
    )(acc, sn, a1)


def kernel(x, adj_indices, adj_values, W, a1, a2):
    row = adj_indices[0].astype(jnp.int32)
    col = adj_indices[1].astype(jnp.int32)
    val = adj_values.astype(jnp.float32)
    cv = jnp.stack(
        [col.reshape(-1, CHUNK),
         row.reshape(-1, CHUNK),
         lax.bitcast_convert_type(val, jnp.int32).reshape(-1, CHUNK)],
        axis=1).reshape(-1)
    tab, rd = _stage1(x, W, a2)
    rd1 = jnp.pad(rd[:, 0], (0, NPAD - N))
    acc, sn = _sc_kernel(tab, rd1, cv)
    return _stage3(acc, sn, a1)[:N]


# speedup vs baseline: 20.3804x; 1.2273x over previous
import functools

import jax
import jax.numpy as jnp
from jax import lax
from jax.experimental import pallas as pl
from jax.experimental.pallas import tpu as pltpu
from jax.experimental.pallas import tpu_sc as plsc

N = 10000
E = 320000
D = 128
NC = 2
NS = 16
NW = NC * NS
EPW = E // NW
CHUNK = 80
NCHUNK = EPW // CHUNK
SBC = 25
NSB = NCHUNK // SBC
NBUF = 4
NROUND = (SBC - 1) // NBUF
CVW = 3 * CHUNK
NPAD = 10240
ROWS_PER_TILE = NPAD // NS
SNR = NPAD // D


def _stage1_body(x_ref, w_ref, a2_ref, tab_ref, rd_ref):
    s = jnp.dot(x_ref[...], w_ref[...], preferred_element_type=jnp.float32)
    rd = jax.nn.sigmoid(
        jnp.dot(s, a2_ref[...], preferred_element_type=jnp.float32))
    tab_ref[...] = rd * s
    rd_ref[...] = rd


def _stage1(x, W, a2):
    B = 2000
    return pl.pallas_call(
        _stage1_body,
        grid=(N // B,),
        in_specs=[
            pl.BlockSpec((B, D), lambda i: (i, 0)),
            pl.BlockSpec((D, D), lambda i: (0, 0)),
            pl.BlockSpec((D, 1), lambda i: (0, 0)),
        ],
        out_specs=[
            pl.BlockSpec((B, D), lambda i: (i, 0)),
            pl.BlockSpec((B, 1), lambda i: (i, 0)),
        ],
        out_shape=[
            jax.ShapeDtypeStruct((N, D), jnp.float32),
            jax.ShapeDtypeStruct((N, 1), jnp.float32),
        ],
    )(x, W, a2)


def _sc_body(tab_hbm, rd2_hbm, cv_hbm, out_hbm, sn_hbm,
             cv_s, rowv0, rowv1, rowv2, rowv3, rows0, rows1, rows2, rows3,
             acc, sg0, sg1, sg2, sg3, ss0, ss1, ss2, ss3):
    c = lax.axis_index("c")
    s = lax.axis_index("s")
    w = s * NC + c

    rows_b = [rows0, rows1, rows2, rows3]
    rowv_b = [rowv0, rowv1, rowv2, rowv3]
    sg = [sg0, sg1, sg2, sg3]
    ss = [ss0, ss1, ss2, ss3]

    zf16 = jnp.zeros((16,), jnp.float32)

    def zero_body(i, carry):
        for j in range(D // 16):
            rows0[i, pl.ds(j * 16, 16)] = zf16
        return carry
    lax.fori_loop(0, CHUNK, zero_body, 0, unroll=False)
    for b in range(ROWS_PER_TILE // CHUNK):
        pltpu.async_copy(
            rows0, acc.at[pl.ds(s * ROWS_PER_TILE + b * CHUNK, CHUNK)], ss0)
    for b in range(ROWS_PER_TILE // CHUNK):
        pltpu.make_async_copy(
            rows0, acc.at[pl.ds(s * ROWS_PER_TILE + b * CHUNK, CHUNK)],
            ss0).wait()
    plsc.subcore_barrier()

    def start_gather(kk, b):
        pltpu.async_copy(tab_hbm.at[cv_s.at[pl.ds(kk * CVW, CHUNK)]],
                         rows_b[b], sg[b])

    def wait_gather(kk, b):
        pltpu.make_async_copy(tab_hbm.at[cv_s.at[pl.ds(kk * CVW, CHUNK)]],
                              rows_b[b], sg[b]).wait()

    def start_scatter(b):
        pltpu.async_copy(rows_b[b], acc.at[rowv_b[b]], ss[b], add=True)

    def wait_scatter(b):
        pltpu.make_async_copy(rows_b[b], acc.at[rowv_b[b]], ss[b]).wait()

    def process(kk, b):
        def group_body(g, carry2):
            base = kk * CVW + g * 16
            row16 = cv_s[pl.ds(base + CHUNK, 16)]
            val16 = plsc.bitcast(cv_s[pl.ds(base + 2 * CHUNK, 16)],
                                 jnp.float32)
            rowv_b[b][pl.ds(g * 16, 16)] = row16
            for l in range(16):
                v = val16[l]
                e = g * 16 + l
                for j in range(D // 16):
                    sl = pl.ds(j * 16, 16)
                    rows_b[b][e, sl] = rows_b[b][e, sl] * v
            return carry2

        lax.fori_loop(0, CHUNK // 16, group_body, 0, unroll=False)

    def stage_cv(sb):
        pltpu.sync_copy(cv_hbm.at[pl.ds((w * NSB + sb) * (SBC * CVW),
                                        SBC * CVW)], cv_s)

    def sb_body(sb, carry):
        stage_cv(sb)
        start_gather(0, 0)
        start_gather(1, 1)

        def round_body(r, carry2):
            for b in range(NBUF):
                k = r * NBUF + b
                bp = (b + 2) % NBUF

                @pl.when(k >= 2)
                def _():
                    wait_scatter(bp)

                @pl.when(k <= SBC - 3)
                def _():
                    start_gather(k + 2, bp)

                wait_gather(k, b)
                process(k, b)
                start_scatter(b)
            return carry2

        lax.fori_loop(0, NROUND, round_body, 0, unroll=False)

        wait_scatter(2)
        wait_gather(SBC - 1, 0)
        process(SBC - 1, 0)
        start_scatter(0)
        wait_scatter(3)
        wait_scatter(0)
        return carry

    lax.fori_loop(0, NSB, sb_body, 0, unroll=False)

    def zero_sn(i, carry):
        for j in range(D // 16):
            rows1[i, pl.ds(j * 16, 16)] = zf16
        return carry
    lax.fori_loop(0, CHUNK, zero_sn, 0, unroll=False)
    pltpu.sync_copy(rd2_hbm, rows0)

    def sb2_body(sb, carry):
        stage_cv(sb)

        def chunk2(kk, carry2):
            def group2(g, carry3):
                base = kk * CVW + g * 16
                col16 = cv_s[pl.ds(base, 16)]
                row16 = cv_s[pl.ds(base + CHUNK, 16)]
                val16 = plsc.bitcast(cv_s[pl.ds(base + 2 * CHUNK, 16)],
                                     jnp.float32)
                rd16 = plsc.load_gather(
                    rows0,
                    [lax.shift_right_logical(col16, 7),
                     lax.bitwise_and(col16, 127)])
                plsc.addupdate_scatter(
                    rows1,
                    [lax.shift_right_logical(row16, 7),
                     lax.bitwise_and(row16, 127)],
                    val16 * rd16)
                return carry3

            return lax.fori_loop(0, CHUNK // 16, group2, carry2,
                                 unroll=False)

        lax.fori_loop(0, SBC, chunk2, 0, unroll=False)
        return carry

    lax.fori_loop(0, NSB, sb2_body, 0, unroll=False)

    pltpu.sync_copy(rows1, sn_hbm.at[w])
    plsc.subcore_barrier()

    pltpu.sync_copy(acc.at[pl.ds(s * ROWS_PER_TILE, ROWS_PER_TILE)],
                    out_hbm.at[c, pl.ds(s * ROWS_PER_TILE, ROWS_PER_TILE)])


_sc_kernel = functools.partial(
    pl.kernel,
    out_type=[
        jax.ShapeDtypeStruct((NC, NPAD, D), jnp.float32),
        jax.ShapeDtypeStruct((NW, SNR, D), jnp.float32),
    ],
    mesh=plsc.VectorSubcoreMesh(core_axis_name="c", subcore_axis_name="s"),
    compiler_params=pltpu.CompilerParams(needs_layout_passes=False),
    scratch_types=[
        pltpu.VMEM((SBC * CVW,), jnp.int32),
        pltpu.VMEM((CHUNK,), jnp.int32),
        pltpu.VMEM((CHUNK,), jnp.int32),
        pltpu.VMEM((CHUNK,), jnp.int32),
        pltpu.VMEM((CHUNK,), jnp.int32),
        pltpu.VMEM((CHUNK, D), jnp.float32),
        pltpu.VMEM((CHUNK, D), jnp.float32),
        pltpu.VMEM((CHUNK, D), jnp.float32),
        pltpu.VMEM((CHUNK, D), jnp.float32),
        pltpu.VMEM_SHARED((NPAD, D), jnp.float32),
        pltpu.SemaphoreType.DMA,
        pltpu.SemaphoreType.DMA,
        pltpu.SemaphoreType.DMA,
        pltpu.SemaphoreType.DMA,
        pltpu.SemaphoreType.DMA,
        pltpu.SemaphoreType.DMA,
        pltpu.SemaphoreType.DMA,
        pltpu.SemaphoreType.DMA,
    ],
)(_sc_body)


_B3 = 2048


def _stage3_body(acc_ref, sn_ref, a1_ref, out_ref):
    a = acc_ref[0] + acc_ref[1]
    sn_sum = jnp.sum(sn_ref[...], axis=0)
    rows = lax.broadcasted_iota(jnp.int32, (_B3, _B3 // D), 0)
    cols = lax.broadcasted_iota(jnp.int32, (_B3, _B3 // D), 1)
    b1 = (rows // D == cols).astype(jnp.float32)
    tmp = jnp.dot(b1, sn_sum, preferred_element_type=jnp.float32)
    rows2 = lax.broadcasted_iota(jnp.int32, (_B3, D), 0)
    cols2 = lax.broadcasted_iota(jnp.int32, (_B3, D), 1)
    sel = (rows2 % D == cols2).astype(jnp.float32)
    snb = jnp.dot(tmp * sel, jnp.ones((D, D), jnp.float32),
                  preferred_element_type=jnp.float32)
    out = a / snb
    l_d = jax.nn.softplus(
        jnp.dot(out, a1_ref[...], preferred_element_type=jnp.float32))
    out = jnp.maximum(out, 0.0) + l_d * jnp.minimum(out, 0.0)
    out_ref[...] = jnp.where(out > 0, out,
                             jnp.exp(jnp.minimum(out, 0.0)) - 1.0)


def _stage3(acc, sn, a1):
    return pl.pallas_call(
        _stage3_body,
        grid=(NPAD // _B3,),
        in_specs=[
            pl.BlockSpec((NC, _B3, D), lambda i: (0, i, 0)),
            pl.BlockSpec((NW, _B3 // D, D), lambda i: (0, i, 0)),
            pl.BlockSpec((D, 1), lambda i: (0, 0)),
        ],
        out_specs=pl.BlockSpec((_B3, D), lambda i: (i, 0)),
        out_shape=jax.ShapeDtypeStruct((NPAD, D), jnp.float32),
    )(acc, sn, a1)


def kernel(x, adj_indices, adj_values, W, a1, a2):
    row = adj_indices[0].astype(jnp.int32)
    col = adj_indices[1].astype(jnp.int32)
    val = adj_values.astype(jnp.float32)
    cv = jnp.stack(
        [col.reshape(-1, CHUNK),
         row.reshape(-1, CHUNK),
         lax.bitcast_convert_type(val, jnp.int32).reshape(-1, CHUNK)],
        axis=1).reshape(-1)
    tab, rd = _stage1(x, W, a2)
    rd2 = jnp.pad(rd[:, 0], (0, NPAD - N)).reshape(SNR, D)
    acc, sn = _sc_kernel(tab, rd2, cv)
    return _stage3(acc, sn, a1)[:N]


# speedup vs baseline: 22.7195x; 1.1148x over previous
import functools

import jax
import jax.numpy as jnp
from jax import lax
from jax.experimental import pallas as pl
from jax.experimental.pallas import tpu as pltpu
from jax.experimental.pallas import tpu_sc as plsc

N = 10000
E = 320000
D = 128
NC = 2
NS = 16
NW = NC * NS
EPW = E // NW
CHUNK = 80
NCHUNK = EPW // CHUNK
SBC = 25
NSB = NCHUNK // SBC
NBUF = 4
NROUND = (SBC - 1) // NBUF
CVW = 3 * CHUNK
NPAD = 10240
ROWS_PER_TILE = NPAD // NS
SNR = NPAD // D


def _stage1_body(x_ref, w_ref, a2_ref, tab_ref, rd_ref):
    s = jnp.dot(x_ref[...], w_ref[...], preferred_element_type=jnp.float32)
    rd = jax.nn.sigmoid(
        jnp.dot(s, a2_ref[...], preferred_element_type=jnp.float32))
    tab_ref[...] = rd * s
    rd_ref[...] = rd


def _stage1(x, W, a2):
    B = 2000
    return pl.pallas_call(
        _stage1_body,
        grid=(N // B,),
        in_specs=[
            pl.BlockSpec((B, D), lambda i: (i, 0)),
            pl.BlockSpec((D, D), lambda i: (0, 0)),
            pl.BlockSpec((D, 1), lambda i: (0, 0)),
        ],
        out_specs=[
            pl.BlockSpec((B, D), lambda i: (i, 0)),
            pl.BlockSpec((B, 1), lambda i: (i, 0)),
        ],
        out_shape=[
            jax.ShapeDtypeStruct((N, D), jnp.float32),
            jax.ShapeDtypeStruct((N, 1), jnp.float32),
        ],
    )(x, W, a2)


def _sc_body(tab_hbm, rd2_hbm, col_hbm, row_hbm, val_hbm, out_hbm, sn_hbm,
             col_s, row_s, val_s, rowv0, rowv1, rowv2, rowv3,
             rows0, rows1, rows2, rows3,
             acc, sg0, sg1, sg2, sg3, ss0, ss1, ss2, ss3):
    c = lax.axis_index("c")
    s = lax.axis_index("s")
    w = s * NC + c

    rows_b = [rows0, rows1, rows2, rows3]
    rowv_b = [rowv0, rowv1, rowv2, rowv3]
    sg = [sg0, sg1, sg2, sg3]
    ss = [ss0, ss1, ss2, ss3]

    zf16 = jnp.zeros((16,), jnp.float32)

    def zero_body(i, carry):
        for j in range(D // 16):
            rows0[i, pl.ds(j * 16, 16)] = zf16
        return carry
    lax.fori_loop(0, CHUNK, zero_body, 0, unroll=False)
    for b in range(ROWS_PER_TILE // CHUNK):
        pltpu.async_copy(
            rows0, acc.at[pl.ds(s * ROWS_PER_TILE + b * CHUNK, CHUNK)], ss0)
    for b in range(ROWS_PER_TILE // CHUNK):
        pltpu.make_async_copy(
            rows0, acc.at[pl.ds(s * ROWS_PER_TILE + b * CHUNK, CHUNK)],
            ss0).wait()
    plsc.subcore_barrier()

    def start_gather(kk, b):
        pltpu.async_copy(tab_hbm.at[col_s.at[pl.ds(kk * CHUNK, CHUNK)]],
                         rows_b[b], sg[b])

    def wait_gather(kk, b):
        pltpu.make_async_copy(tab_hbm.at[col_s.at[pl.ds(kk * CHUNK, CHUNK)]],
                              rows_b[b], sg[b]).wait()

    def start_scatter(b):
        pltpu.async_copy(rows_b[b], acc.at[rowv_b[b]], ss[b], add=True)

    def wait_scatter(b):
        pltpu.make_async_copy(rows_b[b], acc.at[rowv_b[b]], ss[b]).wait()

    def process(kk, b):
        def group_body(g, carry2):
            base = kk * CHUNK + g * 16
            row16 = row_s[pl.ds(base, 16)]
            val16 = val_s[pl.ds(base, 16)]
            rowv_b[b][pl.ds(g * 16, 16)] = row16
            for l in range(16):
                v = val16[l]
                e = g * 16 + l
                for j in range(D // 16):
                    sl = pl.ds(j * 16, 16)
                    rows_b[b][e, sl] = rows_b[b][e, sl] * v
            return carry2

        lax.fori_loop(0, CHUNK // 16, group_body, 0, unroll=False)

    def stage_cv(sb):
        base = w * EPW + sb * (SBC * CHUNK)
        pltpu.sync_copy(col_hbm.at[pl.ds(base, SBC * CHUNK)], col_s)
        pltpu.sync_copy(row_hbm.at[pl.ds(base, SBC * CHUNK)], row_s)
        pltpu.sync_copy(val_hbm.at[pl.ds(base, SBC * CHUNK)], val_s)

    def sb_body(sb, carry):
        stage_cv(sb)
        start_gather(0, 0)
        start_gather(1, 1)

        def round_body(r, carry2):
            for b in range(NBUF):
                k = r * NBUF + b
                bp = (b + 2) % NBUF

                @pl.when(k >= 2)
                def _():
                    wait_scatter(bp)

                @pl.when(k <= SBC - 3)
                def _():
                    start_gather(k + 2, bp)

                wait_gather(k, b)
                process(k, b)
                start_scatter(b)
            return carry2

        lax.fori_loop(0, NROUND, round_body, 0, unroll=False)

        wait_scatter(2)
        wait_gather(SBC - 1, 0)
        process(SBC - 1, 0)
        start_scatter(0)
        wait_scatter(3)
        wait_scatter(0)
        return carry

    lax.fori_loop(0, NSB, sb_body, 0, unroll=False)

    def zero_sn(i, carry):
        for j in range(D // 16):
            rows1[i, pl.ds(j * 16, 16)] = zf16
        return carry
    lax.fori_loop(0, CHUNK, zero_sn, 0, unroll=False)
    pltpu.sync_copy(rd2_hbm, rows0)

    def sb2_body(sb, carry):
        stage_cv(sb)

        def chunk2(kk, carry2):
            def group2(g, carry3):
                base = kk * CHUNK + g * 16
                col16 = col_s[pl.ds(base, 16)]
                row16 = row_s[pl.ds(base, 16)]
                val16 = val_s[pl.ds(base, 16)]
                rd16 = plsc.load_gather(
                    rows0,
                    [lax.shift_right_logical(col16, 7),
                     lax.bitwise_and(col16, 127)])
                plsc.addupdate_scatter(
                    rows1,
                    [lax.shift_right_logical(row16, 7),
                     lax.bitwise_and(row16, 127)],
                    val16 * rd16)
                return carry3

            return lax.fori_loop(0, CHUNK // 16, group2, carry2,
                                 unroll=False)

        lax.fori_loop(0, SBC, chunk2, 0, unroll=False)
        return carry

    lax.fori_loop(0, NSB, sb2_body, 0, unroll=False)

    pltpu.sync_copy(rows1, sn_hbm.at[w])
    plsc.subcore_barrier()

    pltpu.sync_copy(acc.at[pl.ds(s * ROWS_PER_TILE, ROWS_PER_TILE)],
                    out_hbm.at[c, pl.ds(s * ROWS_PER_TILE, ROWS_PER_TILE)])


_sc_kernel = functools.partial(
    pl.kernel,
    out_type=[
        jax.ShapeDtypeStruct((NC, NPAD, D), jnp.float32),
        jax.ShapeDtypeStruct((NW, SNR, D), jnp.float32),
    ],
    mesh=plsc.VectorSubcoreMesh(core_axis_name="c", subcore_axis_name="s"),
    compiler_params=pltpu.CompilerParams(needs_layout_passes=False),
    scratch_types=[
        pltpu.VMEM((SBC * CHUNK,), jnp.int32),
        pltpu.VMEM((SBC * CHUNK,), jnp.int32),
        pltpu.VMEM((SBC * CHUNK,), jnp.float32),
        pltpu.VMEM((CHUNK,), jnp.int32),
        pltpu.VMEM((CHUNK,), jnp.int32),
        pltpu.VMEM((CHUNK,), jnp.int32),
        pltpu.VMEM((CHUNK,), jnp.int32),
        pltpu.VMEM((CHUNK, D), jnp.float32),
        pltpu.VMEM((CHUNK, D), jnp.float32),
        pltpu.VMEM((CHUNK, D), jnp.float32),
        pltpu.VMEM((CHUNK, D), jnp.float32),
        pltpu.VMEM_SHARED((NPAD, D), jnp.float32),
        pltpu.SemaphoreType.DMA,
        pltpu.SemaphoreType.DMA,
        pltpu.SemaphoreType.DMA,
        pltpu.SemaphoreType.DMA,
        pltpu.SemaphoreType.DMA,
        pltpu.SemaphoreType.DMA,
        pltpu.SemaphoreType.DMA,
        pltpu.SemaphoreType.DMA,
    ],
)(_sc_body)


_B3 = 2048


def _stage3_body(acc_ref, sn_ref, a1_ref, out_ref):
    a = acc_ref[0] + acc_ref[1]
    sn_sum = jnp.sum(sn_ref[...], axis=0)
    rows = lax.broadcasted_iota(jnp.int32, (_B3, _B3 // D), 0)
    cols = lax.broadcasted_iota(jnp.int32, (_B3, _B3 // D), 1)
    b1 = (rows // D == cols).astype(jnp.float32)
    tmp = jnp.dot(b1, sn_sum, preferred_element_type=jnp.float32)
    rows2 = lax.broadcasted_iota(jnp.int32, (_B3, D), 0)
    cols2 = lax.broadcasted_iota(jnp.int32, (_B3, D), 1)
    sel = (rows2 % D == cols2).astype(jnp.float32)
    snb = jnp.dot(tmp * sel, jnp.ones((D, D), jnp.float32),
                  preferred_element_type=jnp.float32)
    out = a / snb
    l_d = jax.nn.softplus(
        jnp.dot(out, a1_ref[...], preferred_element_type=jnp.float32))
    out = jnp.maximum(out, 0.0) + l_d * jnp.minimum(out, 0.0)
    out_ref[...] = jnp.where(out > 0, out,
                             jnp.exp(jnp.minimum(out, 0.0)) - 1.0)


def _stage3(acc, sn, a1):
    return pl.pallas_call(
        _stage3_body,
        grid=(NPAD // _B3,),
        in_specs=[
            pl.BlockSpec((NC, _B3, D), lambda i: (0, i, 0)),
            pl.BlockSpec((NW, _B3 // D, D), lambda i: (0, i, 0)),
            pl.BlockSpec((D, 1), lambda i: (0, 0)),
        ],
        out_specs=pl.BlockSpec((_B3, D), lambda i: (i, 0)),
        out_shape=jax.ShapeDtypeStruct((NPAD, D), jnp.float32),
    )(acc, sn, a1)


def kernel(x, adj_indices, adj_values, W, a1, a2):
    row = adj_indices[0].astype(jnp.int32)
    col = adj_indices[1].astype(jnp.int32)
    val = adj_values.astype(jnp.float32)
    tab, rd = _stage1(x, W, a2)
    rd2 = jnp.pad(rd[:, 0], (0, NPAD - N)).reshape(SNR, D)
    acc, sn = _sc_kernel(tab, rd2, col, row, val)
    return _stage3(acc, sn, a1)[:N]
